# baseline (speedup 1.0000x reference)
import functools

import jax
import jax.numpy as jnp
from jax import lax
from jax.experimental import pallas as pl
from jax.experimental.pallas import tpu as pltpu
from jax.experimental.pallas import tpu_sc as plsc

DIM = 32
NUM_CORES = 2
NUM_SUBCORES = 16
NUM_WORKERS = NUM_CORES * NUM_SUBCORES
RB = 4
S = RB * 128
W = 1024

_SC_PARAMS = pltpu.CompilerParams(
    use_tc_tiling_on_sc=False,
    needs_layout_passes=False,
    disable_bounds_checks=True,
)

_MESH = plsc.VectorSubcoreMesh(
    core_axis_name="c",
    subcore_axis_name="s",
    num_cores=NUM_CORES,
    num_subcores=NUM_SUBCORES,
)


@jax.jit
def _transpose_sc(table_cm, last_row):
    C, RV = table_cm.shape
    RP = (RV + 7) & ~7
    per_tile = ((RV // NUM_WORKERS) + 7) & ~7
    r_last = ((RV - W) // 8) * 8

    @functools.partial(
        pl.kernel,
        mesh=_MESH,
        compiler_params=_SC_PARAMS,
        out_type=jax.ShapeDtypeStruct((RP * DIM,), jnp.float32),
        scratch_types=[
            pltpu.VMEM((DIM, W), jnp.float32),
            pltpu.VMEM((W * DIM,), jnp.float32),
            pltpu.VMEM((DIM,), jnp.float32),
        ],
    )
    def k(src_hbm, last_hbm, out_hbm, colbuf, rowbuf, lastbuf):
        wid = lax.axis_index("s") * NUM_CORES + lax.axis_index("c")
        iota = lax.iota(jnp.int32, 16)
        iota32 = iota * DIM
        n_chunks = (per_tile + W - 1) // W

        def chunk_body(j, carry):
            r0 = jnp.minimum(wid * per_tile + j * W, r_last)
            pltpu.sync_copy(src_hbm.at[:, pl.ds(r0, W)], colbuf)

            @plsc.parallel_loop(0, W // 16, unroll=4)
            def wblock(v):
                w0 = v * 16
                wvec = iota + w0
                wbase = iota32 + w0 * DIM
                for k_ in range(DIM):
                    c = (iota + k_) & 31
                    val = plsc.load_gather(colbuf, [c, wvec])
                    plsc.store_scatter(rowbuf, [wbase + c], val)

            pltpu.sync_copy(rowbuf, out_hbm.at[pl.ds(r0 * DIM, W * DIM)])
            return carry

        lax.fori_loop(0, n_chunks, chunk_body, 0)

        @pl.when(wid == 0)
        def _():
            pltpu.sync_copy(last_hbm, lastbuf)
            pltpu.sync_copy(lastbuf, out_hbm.at[pl.ds((RV - 1) * DIM, DIM)])

    return k(table_cm, last_row)


@jax.jit
def _gather_sc(ids_l, table_rm):
    B = ids_l.shape[0]
    R = 16384
    NL = B // R
    n_rh = R // 128
    U = (NL * (n_rh // RB)) // NUM_WORKERS

    @functools.partial(
        pl.kernel,
        mesh=_MESH,
        compiler_params=_SC_PARAMS,
        out_type=jax.ShapeDtypeStruct((NL * 4 * n_rh * 8 * 128,), jnp.float32),
        scratch_types=[
            pltpu.VMEM((B // NUM_WORKERS,), jnp.int32),
            pltpu.VMEM((S, DIM), jnp.float32),
            pltpu.VMEM((S, DIM), jnp.float32),
            pltpu.VMEM((4 * RB * 8 * 128,), jnp.float32),
            pltpu.VMEM((4 * RB * 8 * 128,), jnp.float32),
            pltpu.SemaphoreType.DMA,
            pltpu.SemaphoreType.DMA,
            pltpu.SemaphoreType.DMA,
            pltpu.SemaphoreType.DMA,
        ],
    )
    def k(idx_hbm, table_hbm, out_hbm, idx_all, rows0, rows1,
          t0, t1, gsem0, gsem1, osem0, osem1):
        rows_b = (rows0, rows1)
        t_b = (t0, t1)
        gsem_b = (gsem0, gsem1)
        osem_b = (osem0, osem1)

        wid = lax.axis_index("s") * NUM_CORES + lax.axis_index("c")
        u0 = wid * U
        iota = lax.iota(jnp.int32, 16)

        def out_off(u, ch):
            l = u // (n_rh // RB)
            rhb = u % (n_rh // RB)
            return ((l * 4 + ch) * n_rh + rhb * RB) * 1024

        pltpu.sync_copy(idx_hbm.at[pl.ds(u0 * S, U * S)], idx_all)

        def fire(i, b):
            pltpu.async_copy(
                table_hbm.at[idx_all.at[pl.ds(i * S, S)]],
                rows_b[b], gsem_b[b])

        def transpose_unit(b):
            rows, t = rows_b[b], t_b[b]

            @plsc.parallel_loop(0, S // 16, unroll=4)
            def sblock(v):
                s0 = v * 16
                svec = iota + s0
                soff = (s0 // 128) * 1024 + (s0 % 128) + iota
                for k_ in range(DIM):
                    c = (iota + k_) & 31
                    caddr = ((c & 24) << 9) + ((c & 7) << 7)
                    val = plsc.load_gather(rows, [svec, c])
                    plsc.store_scatter(t, [caddr + soff], val)

        def store_unit(u, b):
            for ch in range(4):
                pltpu.async_copy(
                    t_b[b].at[pl.ds(ch * RB * 1024, RB * 1024)],
                    out_hbm.at[pl.ds(out_off(u, ch), RB * 1024)],
                    osem_b[b])

        def drain_unit(u, b):
            for ch in range(4):
                pltpu.make_async_copy(
                    t_b[b].at[pl.ds(ch * RB * 1024, RB * 1024)],
                    out_hbm.at[pl.ds(out_off(u, ch), RB * 1024)],
                    osem_b[b]).wait()

        fire(0, 0)

        def body(o, carry):
            for b in range(2):
                i = 2 * o + b
                u = u0 + i

                @pl.when(i + 1 < U)
                def _():
                    fire(i + 1, 1 - b)

                pltpu.make_async_copy(
                    table_hbm.at[idx_all.at[pl.ds(i * S, S)]],
                    rows_b[b], gsem_b[b]).wait()

                @pl.when(i >= 2)
                def _():
                    drain_unit(u - 2, b)

                transpose_unit(b)
                store_unit(u, b)
            return carry

        lax.fori_loop(0, U // 2, body, 0)

        for b in range(2):
            drain_unit(u0 + U - 2 + b, b)

    return k(ids_l, table_rm)


def kernel(inputs, embeddings):
    R, NL = inputs.shape
    ids_l = jnp.reshape(jnp.transpose(inputs), (-1,)).astype(jnp.int32)
    table_cm = jnp.transpose(embeddings)
    last_row = embeddings[embeddings.shape[0] - 1]
    n_pad_rows = (embeddings.shape[0] + 7) & ~7
    table_rm = jnp.reshape(_transpose_sc(table_cm, last_row),
                           (n_pad_rows, DIM))
    out5 = _gather_sc(ids_l, table_rm)
    out = jnp.reshape(out5, (NL, 4, R // 128, 8, 128))
    out = jnp.transpose(out, (2, 4, 0, 1, 3))
    return jnp.reshape(out, (R, NL, DIM))


# speedup vs baseline: 4.8006x; 4.8006x over previous
import functools

import jax
import jax.numpy as jnp
from jax import lax
from jax.experimental import pallas as pl
from jax.experimental.pallas import tpu as pltpu
from jax.experimental.pallas import tpu_sc as plsc

DIM = 32
NUM_CORES = 2
NUM_SUBCORES = 16
NUM_WORKERS = NUM_CORES * NUM_SUBCORES
RB = 4
S = RB * 128


@jax.jit
def _gather_sc(ids_l, embeddings):
    B = ids_l.shape[0]
    R = 16384
    NL = B // R
    n_rh = R // 128
    U = (NL * (n_rh // RB)) // NUM_WORKERS
    mesh = plsc.VectorSubcoreMesh(
        core_axis_name="c",
        subcore_axis_name="s",
        num_cores=NUM_CORES,
        num_subcores=NUM_SUBCORES,
    )

    @functools.partial(
        pl.kernel,
        mesh=mesh,
        compiler_params=pltpu.CompilerParams(
            use_tc_tiling_on_sc=False,
            needs_layout_passes=False,
            disable_bounds_checks=True,
        ),
        out_type=jax.ShapeDtypeStruct((NL * 4 * n_rh * 8 * 128,), jnp.float32),
        scratch_types=[
            pltpu.VMEM((B // NUM_WORKERS,), jnp.int32),
            pltpu.VMEM((S, DIM), jnp.float32),
            pltpu.VMEM((S, DIM), jnp.float32),
            pltpu.VMEM((4 * RB * 8 * 128,), jnp.float32),
            pltpu.VMEM((4 * RB * 8 * 128,), jnp.float32),
            pltpu.SemaphoreType.DMA,
            pltpu.SemaphoreType.DMA,
            pltpu.SemaphoreType.DMA,
            pltpu.SemaphoreType.DMA,
        ],
    )
    def k(idx_hbm, table_hbm, out_hbm, idx_all, rows0, rows1,
          t0, t1, gsem0, gsem1, osem0, osem1):
        rows_b = (rows0, rows1)
        t_b = (t0, t1)
        gsem_b = (gsem0, gsem1)
        osem_b = (osem0, osem1)

        wid = lax.axis_index("s") * NUM_CORES + lax.axis_index("c")
        u0 = wid * U
        iota = lax.iota(jnp.int32, 16)

        def out_off(u, ch):
            l = u // (n_rh // RB)
            rhb = u % (n_rh // RB)
            return ((l * 4 + ch) * n_rh + rhb * RB) * 1024

        pltpu.sync_copy(idx_hbm.at[pl.ds(u0 * S, U * S)], idx_all)

        def fire(i, b):
            pltpu.async_copy(
                table_hbm.at[idx_all.at[pl.ds(i * S, S)]],
                rows_b[b], gsem_b[b])

        def transpose_unit(b):
            rows, t = rows_b[b], t_b[b]

            @plsc.parallel_loop(0, S // 16, unroll=4)
            def sblock(v):
                s0 = v * 16
                svec = iota + s0
                soff = (s0 // 128) * 1024 + (s0 % 128) + iota
                for k_ in range(DIM):
                    c = (iota + k_) & 31
                    caddr = ((c & 24) << 9) + ((c & 7) << 7)
                    val = plsc.load_gather(rows, [svec, c])
                    plsc.store_scatter(t, [caddr + soff], val)

        def store_unit(u, b):
            for ch in range(4):
                pltpu.async_copy(
                    t_b[b].at[pl.ds(ch * RB * 1024, RB * 1024)],
                    out_hbm.at[pl.ds(out_off(u, ch), RB * 1024)],
                    osem_b[b])

        def drain_unit(u, b):
            for ch in range(4):
                pltpu.make_async_copy(
                    t_b[b].at[pl.ds(ch * RB * 1024, RB * 1024)],
                    out_hbm.at[pl.ds(out_off(u, ch), RB * 1024)],
                    osem_b[b]).wait()

        fire(0, 0)

        def body(o, carry):
            for b in range(2):
                i = 2 * o + b
                u = u0 + i

                @pl.when(i + 1 < U)
                def _():
                    fire(i + 1, 1 - b)

                pltpu.make_async_copy(
                    table_hbm.at[idx_all.at[pl.ds(i * S, S)]],
                    rows_b[b], gsem_b[b]).wait()

                @pl.when(i >= 2)
                def _():
                    drain_unit(u - 2, b)

                transpose_unit(b)
                store_unit(u, b)
            return carry

        lax.fori_loop(0, U // 2, body, 0)

        for b in range(2):
            drain_unit(u0 + U - 2 + b, b)

    return k(ids_l, embeddings)


def kernel(inputs, embeddings):
    R, NL = inputs.shape
    ids_l = jnp.reshape(jnp.transpose(inputs), (-1,)).astype(jnp.int32)
    out5 = _gather_sc(ids_l, embeddings)
    out = jnp.reshape(out5, (NL, 4, R // 128, 8, 128))
    out = jnp.transpose(out, (2, 4, 0, 1, 3))
    return jnp.reshape(out, (R, NL, DIM))


# speedup vs baseline: 4.8772x; 1.0160x over previous
import functools

import jax
import jax.numpy as jnp
from jax import lax
from jax.experimental import pallas as pl
from jax.experimental.pallas import tpu as pltpu
from jax.experimental.pallas import tpu_sc as plsc

DIM = 32
NUM_CORES = 2
NUM_SUBCORES = 16
NUM_WORKERS = NUM_CORES * NUM_SUBCORES
RB = 4
S = RB * 128


@jax.jit
def _gather_sc(ids_l, embeddings):
    B = ids_l.shape[0]
    R = 16384
    NL = B // R
    n_rh = R // 128
    U = (NL * (n_rh // RB)) // NUM_WORKERS
    mesh = plsc.VectorSubcoreMesh(
        core_axis_name="c",
        subcore_axis_name="s",
        num_cores=NUM_CORES,
        num_subcores=NUM_SUBCORES,
    )

    @functools.partial(
        pl.kernel,
        mesh=mesh,
        compiler_params=pltpu.CompilerParams(
            use_tc_tiling_on_sc=False,
            needs_layout_passes=False,
            disable_bounds_checks=True,
        ),
        out_type=jax.ShapeDtypeStruct((NL * 4 * n_rh * 8 * 128,), jnp.float32),
        scratch_types=[
            pltpu.VMEM((B // NUM_WORKERS,), jnp.int32),
            pltpu.VMEM((S, DIM), jnp.float32),
            pltpu.VMEM((S, DIM), jnp.float32),
            pltpu.VMEM((4 * RB * 8 * 128,), jnp.float32),
            pltpu.VMEM((4 * RB * 8 * 128,), jnp.float32),
            pltpu.SemaphoreType.DMA,
            pltpu.SemaphoreType.DMA,
            pltpu.SemaphoreType.DMA,
            pltpu.SemaphoreType.DMA,
        ],
    )
    def k(idx_hbm, table_hbm, out_hbm, idx_all, rows0, rows1,
          t0, t1, gsem0, gsem1, osem0, osem1):
        rows_b = (rows0, rows1)
        t_b = (t0, t1)
        gsem_b = (gsem0, gsem1)
        osem_b = (osem0, osem1)

        wid = lax.axis_index("s") * NUM_CORES + lax.axis_index("c")
        u0 = wid * U
        iota = lax.iota(jnp.int32, 16)

        def out_off(u, ch):
            l = u // (n_rh // RB)
            rhb = u % (n_rh // RB)
            return ((l * 4 + ch) * n_rh + rhb * RB) * 1024

        pltpu.sync_copy(idx_hbm.at[pl.ds(u0 * S, U * S)], idx_all)

        def fire(i, b):
            pltpu.async_copy(
                table_hbm.at[idx_all.at[pl.ds(i * S, S)]],
                rows_b[b], gsem_b[b])

        def transpose_unit(b):
            rows, t = rows_b[b], t_b[b]

            @plsc.parallel_loop(0, S // 16, unroll=8)
            def sblock(v):
                s0 = v * 16
                svec = iota + s0
                soff = (s0 // 128) * 1024 + (s0 % 128) + iota
                for k_ in range(DIM):
                    c = (iota + k_) & 31
                    caddr = ((c & 24) << 9) + ((c & 7) << 7)
                    val = plsc.load_gather(rows, [svec, c])
                    plsc.store_scatter(t, [caddr + soff], val)

        def store_unit(u, b):
            for ch in range(4):
                pltpu.async_copy(
                    t_b[b].at[pl.ds(ch * RB * 1024, RB * 1024)],
                    out_hbm.at[pl.ds(out_off(u, ch), RB * 1024)],
                    osem_b[b])

        def drain_unit(u, b):
            for ch in range(4):
                pltpu.make_async_copy(
                    t_b[b].at[pl.ds(ch * RB * 1024, RB * 1024)],
                    out_hbm.at[pl.ds(out_off(u, ch), RB * 1024)],
                    osem_b[b]).wait()

        fire(0, 0)

        def body(o, carry):
            for b in range(2):
                i = 2 * o + b
                u = u0 + i

                @pl.when(i + 1 < U)
                def _():
                    fire(i + 1, 1 - b)

                pltpu.make_async_copy(
                    table_hbm.at[idx_all.at[pl.ds(i * S, S)]],
                    rows_b[b], gsem_b[b]).wait()

                @pl.when(i >= 2)
                def _():
                    drain_unit(u - 2, b)

                transpose_unit(b)
                store_unit(u, b)
            return carry

        lax.fori_loop(0, U // 2, body, 0)

        for b in range(2):
            drain_unit(u0 + U - 2 + b, b)

    return k(ids_l, embeddings)


def kernel(inputs, embeddings):
    R, NL = inputs.shape
    ids_l = jnp.reshape(jnp.transpose(inputs), (-1,)).astype(jnp.int32)
    out5 = _gather_sc(ids_l, embeddings)
    out = jnp.reshape(out5, (NL, 4, R // 128, 8, 128))
    out = jnp.transpose(out, (2, 4, 0, 1, 3))
    return jnp.reshape(out, (R, NL, DIM))
